# Initial kernel scaffold; baseline (speedup 1.0000x reference)
import jax
import jax.numpy as jnp
from jax.experimental import pallas as pl


def kernel(x, edges, membership, params):
    raise NotImplementedError("write your pallas kernel here")


# speedup vs baseline: 1.5987x; 1.5987x over previous
import functools

import jax
import jax.numpy as jnp
from jax import lax
from jax.experimental import pallas as pl
from jax.experimental.pallas import tpu as pltpu
from jax.experimental.pallas import tpu_sc as plsc

N = 4096
E = 65536
D = 128
B = 64
FDIMS = [100, 50, 20]
NBLK = 3
H = 256
OUT = 2

_NC = 2
_NS = 16
_NW = _NC * _NS
_K = 128
_NEG = -3.0e38
_INTERPRET = False


def _leaky(v):
    return jnp.where(v >= 0, v, 0.1 * v)


def _seg_agg(data, idx, ids, off, n_out):
    n_local = n_out // _NW
    mesh = plsc.VectorSubcoreMesh(core_axis_name="c", subcore_axis_name="s")
    oshape = jax.ShapeDtypeStruct((n_out, D), jnp.float32)
    ids_rep = jnp.repeat(ids, 16)
    off_rep = jnp.repeat(off[:_NW + 1], 16)

    @functools.partial(
        pl.kernel,
        mesh=mesh,
        out_type=(oshape, oshape),
        scratch_types=[
            pltpu.VMEM(((_NW + 1) * 16,), jnp.int32),
            pltpu.VMEM((_K,), jnp.int32),
            pltpu.VMEM((_K * 16,), jnp.int32),
            pltpu.VMEM((_K, D), jnp.float32),
            pltpu.VMEM((n_local, D), jnp.float32),
            pltpu.VMEM((n_local, D), jnp.float32),
            pltpu.SemaphoreType.DMA,
        ],
    )
    def k(data_h, idx_h, ids_h, off_h, sum_h, mx_h,
          off_v, idx_v, ids_v, rows_v, asum, amax, sem):
        w = lax.axis_index("s") * _NC + lax.axis_index("c")
        pltpu.sync_copy(off_h, off_v)
        e0 = off_v[pl.ds(w * 16, 16)][0]
        e1 = off_v[pl.ds(w * 16 + 16, 16)][0]
        base = w * n_local

        @pl.loop(0, n_local)
        def _(r):
            for c in range(0, D, 16):
                asum[r, pl.ds(c, 16)] = jnp.zeros((16,), jnp.float32)
                amax[r, pl.ds(c, 16)] = jnp.full((16,), _NEG, jnp.float32)

        lo = (e0 // _K) * _K

        @pl.loop(lo, e1, step=_K)
        def _(cs):
            pltpu.sync_copy(idx_h.at[pl.ds(cs, _K)], idx_v)
            pltpu.sync_copy(ids_h.at[pl.ds(cs * 16, _K * 16)], ids_v)
            pltpu.async_copy(data_h.at[idx_v], rows_v, sem).wait()
            j0 = jnp.maximum(e0, cs) - cs
            j1 = jnp.minimum(e1, cs + _K) - cs

            @pl.loop(j0, j1)
            def _(j):
                d = ids_v[pl.ds(j * 16, 16)][0] - base
                for c in range(0, D, 16):
                    vec = rows_v[j, pl.ds(c, 16)]
                    asum[d, pl.ds(c, 16)] = asum[d, pl.ds(c, 16)] + vec
                    amax[d, pl.ds(c, 16)] = jnp.maximum(
                        amax[d, pl.ds(c, 16)], vec)

        pltpu.sync_copy(asum, sum_h.at[pl.ds(base, n_local)])
        pltpu.sync_copy(amax, mx_h.at[pl.ds(base, n_local)])

    return k(data, idx, ids_rep, off_rep)


def _embed_body(x_ref, tab_ref, out_ref):
    cols = lax.broadcasted_iota(jnp.int32, (N, 256), 1)
    m = jnp.zeros((N, 256), jnp.float32)
    offs = [0, 100, 150]
    for f in range(3):
        xf = x_ref[:, f:f + 1] + offs[f]
        m = m + (cols == xf).astype(jnp.float32)
    out_ref[...] = jnp.dot(m, tab_ref[...],
                           preferred_element_type=jnp.float32, precision=jax.lax.Precision.HIGHEST)


def _embed(x, tab):
    return pl.pallas_call(
        _embed_body,
        out_shape=jax.ShapeDtypeStruct((N, D), jnp.float32),
        interpret=_INTERPRET,
    )(x, tab)


_RB = 256
_NRB = N // _RB


def _gn0(t, g, b):
    m = jnp.mean(t, axis=0, keepdims=True)
    v = jnp.mean((t - m) ** 2, axis=0, keepdims=True)
    return (t - m) * lax.rsqrt(v + 1e-5) * g + b


def _mmT(a, w):
    return jnp.dot(a, w, preferred_element_type=jnp.float32,
                   precision=jax.lax.Precision.HIGHEST)


def _prep_body(nparts, *refs):
    it = iter(refs)
    feats = [next(it) for _ in range(nparts - 1)]
    asum, amx = next(it), next(it)
    WzT, WrT, WnT, UzT, UrT, UnT = (next(it) for _ in range(6))
    bz, br, bn = (next(it) for _ in range(3))
    bn_g, bn_b, bn_WT, bn_bias = (next(it) for _ in range(4))
    qg, qb, qWT, qbias = (next(it) for _ in range(4))
    kg, kb, kWT, kbias = (next(it) for _ in range(4))
    vg, vb, vWT, vbias = (next(it) for _ in range(4))
    q_o, k_o, v_o = (next(it) for _ in range(3))

    h = feats[-1][...]
    mx = amx[...]
    agg = asum[...] + jnp.where(mx > -1e37, mx, 0.0)

    z = jax.nn.sigmoid(_mmT(agg, WzT[...]) + _mmT(h, UzT[...]) + bz[...])
    r = jax.nn.sigmoid(_mmT(agg, WrT[...]) + _mmT(h, UrT[...]) + br[...])
    n = jnp.tanh(_mmT(agg, WnT[...]) + r * _mmT(h, UnT[...]) + bn[...])
    hu = (1.0 - z) * n + z * h + h

    acc = jnp.zeros((N, D), jnp.float32) + bn_bias[...]
    parts = [f[...] for f in feats] + [hu]
    for j, pj in enumerate(parts):
        g = bn_g[:, j * D:(j + 1) * D]
        b = bn_b[:, j * D:(j + 1) * D]
        acc = acc + _mmT(_gn0(pj, g, b), bn_WT[j * D:(j + 1) * D, :])
    x1 = _leaky(acc)
    q_o[...] = _leaky(_mmT(_gn0(x1, qg[...], qb[...]), qWT[...]) + qbias[...])
    k_o[...] = _leaky(_mmT(_gn0(x1, kg[...], kb[...]), kWT[...]) + kbias[...])
    v_o[...] = _leaky(_mmT(_gn0(x1, vg[...], vb[...]), vWT[...]) + vbias[...])


def _att_body(q_r, k_r, v_r, lng, lnb, out):
    i = pl.program_id(0)
    kb_ = k_r[pl.ds(i * _RB, _RB), :]
    s = lax.dot_general(kb_, q_r[...], (((1,), (1,)), ((), ())),
                        preferred_element_type=jnp.float32,
                        precision=jax.lax.Precision.HIGHEST)
    s = s - jnp.max(s, axis=1, keepdims=True)
    e = jnp.exp(s)
    a = e / jnp.sum(e, axis=1, keepdims=True)
    o = _leaky(_mmT(a, v_r[...]))
    m = jnp.mean(o, axis=1, keepdims=True)
    va = jnp.mean((o - m) ** 2, axis=1, keepdims=True)
    out[...] = (o - m) * lax.rsqrt(va + 1e-5) * lng[...] + lnb[...]


def _block_tc(feats, asum, amx, p):
    nparts = len(feats) + 1
    row = lambda a: a.reshape(1, -1)
    ins = list(feats) + [asum, amx] + [
        p['Wz'].T, p['Wr'].T, p['Wn'].T, p['Uz'].T, p['Ur'].T, p['Un'].T,
        row(p['bz']), row(p['br']), row(p['bn']),
        row(p['bn_g']), row(p['bn_b']), p['bn_W'].T, row(p['bn_bias']),
        row(p['q_g']), row(p['q_b']), p['q_W'].T, row(p['q_bias']),
        row(p['k_g']), row(p['k_b']), p['k_W'].T, row(p['k_bias']),
        row(p['v_g']), row(p['v_b']), p['v_W'].T, row(p['v_bias']),
    ]
    osh = jax.ShapeDtypeStruct((N, D), jnp.float32)
    q, k, v = pl.pallas_call(
        functools.partial(_prep_body, nparts),
        out_shape=(osh, osh, osh),
        interpret=_INTERPRET,
    )(*ins)
    in_specs = [pl.BlockSpec((N, D), lambda i: (0, 0))] * 3 + \
               [pl.BlockSpec((1, D), lambda i: (0, 0))] * 2
    return pl.pallas_call(
        _att_body,
        grid=(_NRB,),
        in_specs=in_specs,
        out_specs=pl.BlockSpec((_RB, D), lambda i: (i, 0)),
        out_shape=osh,
        interpret=_INTERPRET,
    )(q, k, v, row(p['ln_g']), row(p['ln_b']))


def _mlp_body(rsum, rmx, d0W, d0b, g0, b0, d1W, d1b, d2W, d2b, out):
    parts = []
    for f in range(4):
        s = rsum[f * B:(f + 1) * B, :]
        m = rmx[f * B:(f + 1) * B, :]
        parts.append(s + jnp.where(m > -1e37, m, 0.0))
    r = jnp.concatenate(parts, axis=1)
    z = jnp.dot(r, d0W[...], preferred_element_type=jnp.float32, precision=jax.lax.Precision.HIGHEST) + d0b[...]
    zm = jnp.mean(z, axis=0, keepdims=True)
    zv = jnp.mean((z - zm) ** 2, axis=0, keepdims=True)
    z = (z - zm) * lax.rsqrt(zv + 1e-5) * g0[...] + b0[...]
    z = jax.nn.relu(z)
    z = jax.nn.relu(jnp.dot(z, d1W[...],
                            preferred_element_type=jnp.float32, precision=jax.lax.Precision.HIGHEST) + d1b[...])
    out[...] = jnp.dot(z, d2W[...],
                       preferred_element_type=jnp.float32, precision=jax.lax.Precision.HIGHEST) + d2b[...]


def _mlp(rsum, rmx, params):
    row = lambda a: a.reshape(1, -1)
    d2W = jnp.zeros((D, D), jnp.float32).at[:, :OUT].set(params['d2_W'].T)
    d2b = jnp.zeros((1, D), jnp.float32).at[:, :OUT].set(params['d2_b'])
    o = pl.pallas_call(
        _mlp_body,
        out_shape=jax.ShapeDtypeStruct((B, D), jnp.float32),
        interpret=_INTERPRET,
    )(rsum, rmx, params['d0_W'].T, row(params['d0_b']),
      row(params['bn0_g']), row(params['bn0_b']),
      params['d1_W'].T, row(params['d1_b']), d2W, d2b)
    return o[:, :OUT]


def _worker_offsets(ids, n_out):
    n_local = n_out // _NW
    bounds = jnp.arange(_NW + 1, dtype=jnp.int32) * n_local
    off = jnp.searchsorted(ids, bounds, side='left').astype(jnp.int32)
    return jnp.pad(off, (0, 48 - (_NW + 1)))


@jax.jit
def kernel(x, edges, membership, params):
    src, dst = edges[0], edges[1]
    order = jnp.argsort(dst)
    src_s = src[order].astype(jnp.int32)
    dst_s = dst[order].astype(jnp.int32)
    eoff = _worker_offsets(dst_s, N)

    tab = jnp.concatenate(params['emb'], axis=0)
    tab = jnp.pad(tab, ((0, 256 - tab.shape[0]), (0, 0)))
    h = _embed(x.astype(jnp.int32), tab)

    feats = [h]
    for i in range(NBLK):
        asum, amx = _seg_agg(feats[-1], src_s, dst_s, eoff, N)
        h = _block_tc(feats, asum, amx, params['blocks'][i])
        feats.append(h)

    fs = jnp.concatenate(feats, axis=0)
    mem = membership.astype(jnp.int32)
    rids = jnp.concatenate([mem + B * f for f in range(4)])
    ridx = jnp.arange(4 * N, dtype=jnp.int32)
    roff = _worker_offsets(rids, 4 * B)
    rsum, rmx = _seg_agg(fs, ridx, rids, roff, 4 * B)

    return _mlp(rsum, rmx, params)
